# Initial kernel scaffold; baseline (speedup 1.0000x reference)
import jax
import jax.numpy as jnp
from jax.experimental import pallas as pl


def kernel(x, edge_index, batch, W1, b1, W2, b2):
    raise NotImplementedError("write your pallas kernel here")


# speedup vs baseline: 5.5123x; 5.5123x over previous
import functools

import jax
import jax.numpy as jnp
from jax import lax
from jax.experimental import pallas as pl
from jax.experimental.pallas import tpu as pltpu
from jax.experimental.pallas import tpu_sc as plsc

N_NODES = 50000
N_EDGES = 800000
IN_DIM = 78
HIDDEN = 128
N_GRAPHS = 1024

NC = 2
NS = 16
CHUNK = 64
GROUP = 4
BLK = 512
NPAD = 50048
ROWS_PER_SUB = NPAD // NS
HALF = 40
E_SUB = 50176
N_GROUPS = E_SUB // (GROUP * CHUNK)
E_PAD = E_SUB * NS


def _sc_agg_body(xpad_hbm, src_hbm, dst_hbm, out_hbm, src_v, dst_v,
                 rows0, rows1, agg_sh, sem0, sem1):
    c = lax.axis_index("c")
    s = lax.axis_index("s")
    base = s * ROWS_PER_SUB
    pltpu.sync_copy(xpad_hbm.at[pl.ds(c * NPAD + base, ROWS_PER_SUB), :],
                    agg_sh.at[pl.ds(base, ROWS_PER_SUB), :])
    plsc.subcore_barrier()

    rows = (rows0, rows1)
    sems = (sem0, sem1)

    def group_body(g, carry):
        pltpu.sync_copy(src_hbm.at[c, s, pl.ds(g * GROUP, GROUP)], src_v)
        pltpu.sync_copy(dst_hbm.at[s, pl.ds(g * GROUP, GROUP)], dst_v)
        copies = [None, None]
        copies[0] = pltpu.async_copy(xpad_hbm.at[src_v.at[0]], rows[0], sems[0])
        for b in range(GROUP):
            nxt = (b + 1) % 2
            if b + 1 < GROUP:
                copies[nxt] = pltpu.async_copy(
                    xpad_hbm.at[src_v.at[b + 1]], rows[nxt], sems[nxt])
            copies[b % 2].wait()
            pltpu.sync_copy(rows[b % 2], agg_sh.at[dst_v.at[b]], add=True)
        return carry

    lax.fori_loop(0, N_GROUPS, group_body, 0)
    plsc.subcore_barrier()
    pltpu.sync_copy(agg_sh.at[pl.ds(base, ROWS_PER_SUB), :],
                    out_hbm.at[c, pl.ds(base, ROWS_PER_SUB), :])


@functools.cache
def _sc_agg():
    return pl.kernel(
        _sc_agg_body,
        out_type=jax.ShapeDtypeStruct((NC, NPAD, HALF), jnp.float32),
        mesh=plsc.VectorSubcoreMesh(core_axis_name="c", subcore_axis_name="s",
                                    num_cores=NC, num_subcores=NS),
        scratch_types=[
            pltpu.VMEM((GROUP, CHUNK), jnp.int32),
            pltpu.VMEM((GROUP, CHUNK), jnp.int32),
            pltpu.VMEM((CHUNK, HALF), jnp.float32),
            pltpu.VMEM((CHUNK, HALF), jnp.float32),
            pltpu.VMEM_SHARED((NPAD, HALF), jnp.float32),
            pltpu.SemaphoreType.DMA,
            pltpu.SemaphoreType.DMA,
        ],
        compiler_params=pltpu.CompilerParams(use_tc_tiling_on_sc=False),
    )


def _tc_mlp_pool_body(lo_ref, hi_ref, bcol_ref, w1a_ref, w1b_ref, b1_ref,
                      w2_ref, b2_ref, out_ref, cnt_ref):
    i = pl.program_id(0)
    nsteps = pl.num_programs(0)

    @pl.when(i == 0)
    def _init():
        out_ref[...] = jnp.zeros_like(out_ref)
        cnt_ref[...] = jnp.zeros_like(cnt_ref)

    lo = lo_ref[0]
    hi = hi_ref[0]
    h1 = jnp.dot(lo, w1a_ref[...], preferred_element_type=jnp.float32)
    h1 += jnp.dot(hi, w1b_ref[...], preferred_element_type=jnp.float32)
    h1 = jnp.maximum(h1 + b1_ref[...], 0.0)
    h2 = jnp.dot(h1, w2_ref[...], preferred_element_type=jnp.float32)
    h2 += b2_ref[...]

    row = i * BLK + lax.broadcasted_iota(jnp.int32, (BLK, 1), 0)
    h2 = jnp.where(row < N_NODES, h2, 0.0)

    b = bcol_ref[...]
    gids = lax.broadcasted_iota(jnp.int32, (BLK, N_GRAPHS), 1)
    onehot = jnp.where((b == gids) & (row < N_NODES), 1.0, 0.0)
    out_ref[...] += lax.dot_general(onehot, h2, (((0,), (0,)), ((), ())),
                                    preferred_element_type=jnp.float32)
    cnt_ref[...] += lax.dot_general(onehot, jnp.ones((BLK, 8), jnp.float32),
                                    (((0,), (0,)), ((), ())),
                                    preferred_element_type=jnp.float32)

    @pl.when(i == nsteps - 1)
    def _final():
        counts = jnp.maximum(cnt_ref[:, 0:1], 1.0)
        out_ref[...] = out_ref[...] / counts


def _tc_mlp_pool(agg3d, bcol, w1a, w1b, b1, w2, b2):
    grid = (NPAD + BLK - 1) // BLK
    return pl.pallas_call(
        _tc_mlp_pool_body,
        grid=(grid,),
        in_specs=[
            pl.BlockSpec((1, BLK, HALF), lambda i: (0, i, 0)),
            pl.BlockSpec((1, BLK, HALF), lambda i: (1, i, 0)),
            pl.BlockSpec((BLK, 1), lambda i: (i, 0)),
            pl.BlockSpec((HALF, HIDDEN), lambda i: (0, 0)),
            pl.BlockSpec((HALF, HIDDEN), lambda i: (0, 0)),
            pl.BlockSpec((1, HIDDEN), lambda i: (0, 0)),
            pl.BlockSpec((HIDDEN, HIDDEN), lambda i: (0, 0)),
            pl.BlockSpec((1, HIDDEN), lambda i: (0, 0)),
        ],
        out_specs=pl.BlockSpec((N_GRAPHS, HIDDEN), lambda i: (0, 0)),
        out_shape=jax.ShapeDtypeStruct((N_GRAPHS, HIDDEN), jnp.float32),
        scratch_shapes=[pltpu.VMEM((N_GRAPHS, 8), jnp.float32)],
    )(agg3d, agg3d, bcol, w1a, w1b, b1, w2, b2)


def kernel(x, edge_index, batch, W1, b1, W2, b2):
    src = edge_index[0].astype(jnp.int32)
    dst = edge_index[1].astype(jnp.int32)

    xa = jnp.zeros((NPAD, HALF), jnp.float32).at[:N_NODES, :].set(x[:, :HALF])
    xb = jnp.zeros((NPAD, HALF), jnp.float32).at[:N_NODES, :IN_DIM - HALF].set(
        x[:, HALF:])
    xpad2 = jnp.concatenate([xa, xb], axis=0)

    pad = jnp.full((E_PAD - N_EDGES,), N_NODES, jnp.int32)
    src_p = jnp.concatenate([src, pad])
    dst_p = jnp.concatenate([dst, pad]).reshape(NS, N_GROUPS * GROUP, CHUNK)
    src2 = jnp.stack([src_p, src_p + NPAD]).reshape(
        NC, NS, N_GROUPS * GROUP, CHUNK)

    agg3d = _sc_agg()(xpad2, src2, dst_p)

    bcol = jnp.full((NPAD, 1), N_GRAPHS, jnp.int32).at[:N_NODES, 0].set(
        batch.astype(jnp.int32))

    w1a = W1[:HALF, :]
    w1b = jnp.zeros((HALF, HIDDEN), jnp.float32).at[:IN_DIM - HALF, :].set(
        W1[HALF:, :])
    return _tc_mlp_pool(agg3d, bcol, w1a, w1b, b1.reshape(1, HIDDEN), W2,
                        b2.reshape(1, HIDDEN))


# speedup vs baseline: 6.8488x; 1.2425x over previous
import functools

import jax
import jax.numpy as jnp
from jax import lax
from jax.experimental import pallas as pl
from jax.experimental.pallas import tpu as pltpu
from jax.experimental.pallas import tpu_sc as plsc

N_NODES = 50000
N_EDGES = 800000
IN_DIM = 78
HIDDEN = 128
N_GRAPHS = 1024

NC = 2
NS = 16
CHUNK = 64
STAGE = 2
BLK = 512
NPAD = 50048
ROWS_PER_SUB = NPAD // NS
HALF = 40
E_SUB = 50176
NST = E_SUB // (STAGE * CHUNK)
N_ITERS = NST // 2
E_PAD = E_SUB * NS


def _sc_agg_body(xpad_hbm, src_hbm, dst_hbm, out_hbm, src_a, src_b, dst_a,
                 dst_b, rows0, rows1, agg_sh, sem_r0, sem_r1, sem_ia, sem_ib):
    c = lax.axis_index("c")
    s = lax.axis_index("s")
    base = s * ROWS_PER_SUB
    pltpu.sync_copy(xpad_hbm.at[pl.ds(c * NPAD + base, ROWS_PER_SUB), :],
                    agg_sh.at[pl.ds(base, ROWS_PER_SUB), :])
    plsc.subcore_barrier()

    def wait_rows(rows_v, sem):
        pltpu.make_async_copy(xpad_hbm.at[src_a.at[0]], rows_v, sem).wait()

    def wait_idx(sv, dv, sem):
        pltpu.make_async_copy(src_hbm.at[c, s, 0], sv, sem).wait()
        pltpu.make_async_copy(dst_hbm.at[s, 0], dv, sem).wait()

    pltpu.sync_copy(src_hbm.at[c, s, 0], src_a)
    pltpu.sync_copy(dst_hbm.at[s, 0], dst_a)
    pltpu.async_copy(src_hbm.at[c, s, 1], src_b, sem_ib)
    pltpu.async_copy(dst_hbm.at[s, 1], dst_b, sem_ib)
    pltpu.async_copy(xpad_hbm.at[src_a.at[0]], rows0, sem_r0)

    def body(i, carry):
        st_a = jnp.minimum(2 * i + 2, NST - 1)
        st_b = jnp.minimum(2 * i + 3, NST - 1)
        pltpu.async_copy(xpad_hbm.at[src_a.at[1]], rows1, sem_r1)
        wait_rows(rows0, sem_r0)
        pltpu.sync_copy(rows0, agg_sh.at[dst_a.at[0]], add=True)
        wait_idx(src_b, dst_b, sem_ib)
        pltpu.async_copy(xpad_hbm.at[src_b.at[0]], rows0, sem_r0)
        wait_rows(rows1, sem_r1)
        pltpu.sync_copy(rows1, agg_sh.at[dst_a.at[1]], add=True)
        pltpu.async_copy(src_hbm.at[c, s, st_a], src_a, sem_ia)
        pltpu.async_copy(dst_hbm.at[s, st_a], dst_a, sem_ia)
        pltpu.async_copy(xpad_hbm.at[src_b.at[1]], rows1, sem_r1)
        wait_rows(rows0, sem_r0)
        pltpu.sync_copy(rows0, agg_sh.at[dst_b.at[0]], add=True)
        wait_idx(src_a, dst_a, sem_ia)
        pltpu.async_copy(xpad_hbm.at[src_a.at[0]], rows0, sem_r0)
        wait_rows(rows1, sem_r1)
        pltpu.sync_copy(rows1, agg_sh.at[dst_b.at[1]], add=True)
        pltpu.async_copy(src_hbm.at[c, s, st_b], src_b, sem_ib)
        pltpu.async_copy(dst_hbm.at[s, st_b], dst_b, sem_ib)
        return carry

    lax.fori_loop(0, N_ITERS, body, 0)
    wait_rows(rows0, sem_r0)
    wait_idx(src_b, dst_b, sem_ib)
    plsc.subcore_barrier()
    pltpu.sync_copy(agg_sh.at[pl.ds(base, ROWS_PER_SUB), :],
                    out_hbm.at[c, pl.ds(base, ROWS_PER_SUB), :])


@functools.cache
def _sc_agg():
    return pl.kernel(
        _sc_agg_body,
        out_type=jax.ShapeDtypeStruct((NC, NPAD, HALF), jnp.float32),
        mesh=plsc.VectorSubcoreMesh(core_axis_name="c", subcore_axis_name="s",
                                    num_cores=NC, num_subcores=NS),
        scratch_types=[
            pltpu.VMEM((STAGE, CHUNK), jnp.int32),
            pltpu.VMEM((STAGE, CHUNK), jnp.int32),
            pltpu.VMEM((STAGE, CHUNK), jnp.int32),
            pltpu.VMEM((STAGE, CHUNK), jnp.int32),
            pltpu.VMEM((CHUNK, HALF), jnp.float32),
            pltpu.VMEM((CHUNK, HALF), jnp.float32),
            pltpu.VMEM_SHARED((NPAD, HALF), jnp.float32),
            pltpu.SemaphoreType.DMA,
            pltpu.SemaphoreType.DMA,
            pltpu.SemaphoreType.DMA,
            pltpu.SemaphoreType.DMA,
        ],
        compiler_params=pltpu.CompilerParams(use_tc_tiling_on_sc=False),
    )


def _tc_mlp_pool_body(lo_ref, hi_ref, bcol_ref, w1a_ref, w1b_ref, b1_ref,
                      w2_ref, b2_ref, out_ref, cnt_ref):
    i = pl.program_id(0)
    nsteps = pl.num_programs(0)

    @pl.when(i == 0)
    def _init():
        out_ref[...] = jnp.zeros_like(out_ref)
        cnt_ref[...] = jnp.zeros_like(cnt_ref)

    lo = lo_ref[0]
    hi = hi_ref[0]
    h1 = jnp.dot(lo, w1a_ref[...], preferred_element_type=jnp.float32)
    h1 += jnp.dot(hi, w1b_ref[...], preferred_element_type=jnp.float32)
    h1 = jnp.maximum(h1 + b1_ref[...], 0.0)
    h2 = jnp.dot(h1, w2_ref[...], preferred_element_type=jnp.float32)
    h2 += b2_ref[...]

    row = i * BLK + lax.broadcasted_iota(jnp.int32, (BLK, 1), 0)
    h2 = jnp.where(row < N_NODES, h2, 0.0)

    b = bcol_ref[...]
    gids = lax.broadcasted_iota(jnp.int32, (BLK, N_GRAPHS), 1)
    onehot = jnp.where((b == gids) & (row < N_NODES), 1.0, 0.0)
    out_ref[...] += lax.dot_general(onehot, h2, (((0,), (0,)), ((), ())),
                                    preferred_element_type=jnp.float32)
    cnt_ref[...] += lax.dot_general(onehot, jnp.ones((BLK, 8), jnp.float32),
                                    (((0,), (0,)), ((), ())),
                                    preferred_element_type=jnp.float32)

    @pl.when(i == nsteps - 1)
    def _final():
        counts = jnp.maximum(cnt_ref[:, 0:1], 1.0)
        out_ref[...] = out_ref[...] / counts


def _tc_mlp_pool(agg3d, bcol, w1a, w1b, b1, w2, b2):
    grid = (NPAD + BLK - 1) // BLK
    return pl.pallas_call(
        _tc_mlp_pool_body,
        grid=(grid,),
        in_specs=[
            pl.BlockSpec((1, BLK, HALF), lambda i: (0, i, 0)),
            pl.BlockSpec((1, BLK, HALF), lambda i: (1, i, 0)),
            pl.BlockSpec((BLK, 1), lambda i: (i, 0)),
            pl.BlockSpec((HALF, HIDDEN), lambda i: (0, 0)),
            pl.BlockSpec((HALF, HIDDEN), lambda i: (0, 0)),
            pl.BlockSpec((1, HIDDEN), lambda i: (0, 0)),
            pl.BlockSpec((HIDDEN, HIDDEN), lambda i: (0, 0)),
            pl.BlockSpec((1, HIDDEN), lambda i: (0, 0)),
        ],
        out_specs=pl.BlockSpec((N_GRAPHS, HIDDEN), lambda i: (0, 0)),
        out_shape=jax.ShapeDtypeStruct((N_GRAPHS, HIDDEN), jnp.float32),
        scratch_shapes=[pltpu.VMEM((N_GRAPHS, 8), jnp.float32)],
    )(agg3d, agg3d, bcol, w1a, w1b, b1, w2, b2)


def kernel(x, edge_index, batch, W1, b1, W2, b2):
    src = edge_index[0].astype(jnp.int32)
    dst = edge_index[1].astype(jnp.int32)

    xa = jnp.zeros((NPAD, HALF), jnp.float32).at[:N_NODES, :].set(x[:, :HALF])
    xb = jnp.zeros((NPAD, HALF), jnp.float32).at[:N_NODES, :IN_DIM - HALF].set(
        x[:, HALF:])
    xpad2 = jnp.concatenate([xa, xb], axis=0)

    pad = jnp.full((E_PAD - N_EDGES,), N_NODES, jnp.int32)
    src_p = jnp.concatenate([src, pad])
    dst_p = jnp.concatenate([dst, pad]).reshape(NS, NST, STAGE, CHUNK)
    src2 = jnp.stack([src_p, src_p + NPAD]).reshape(
        NC, NS, NST, STAGE, CHUNK)

    agg3d = _sc_agg()(xpad2, src2, dst_p)

    bcol = jnp.full((NPAD, 1), N_GRAPHS, jnp.int32).at[:N_NODES, 0].set(
        batch.astype(jnp.int32))

    w1a = W1[:HALF, :]
    w1b = jnp.zeros((HALF, HIDDEN), jnp.float32).at[:IN_DIM - HALF, :].set(
        W1[HALF:, :])
    return _tc_mlp_pool(agg3d, bcol, w1a, w1b, b1.reshape(1, HIDDEN), W2,
                        b2.reshape(1, HIDDEN))


# speedup vs baseline: 7.2767x; 1.0625x over previous
import functools

import jax
import jax.numpy as jnp
from jax import lax
from jax.experimental import pallas as pl
from jax.experimental.pallas import tpu as pltpu
from jax.experimental.pallas import tpu_sc as plsc

N_NODES = 50000
N_EDGES = 800000
IN_DIM = 78
HIDDEN = 128
N_GRAPHS = 1024

NC = 2
NS = 16
CHUNK = 64
STAGE = 2
BLK = 2048
NPAD = 50048
ROWS_PER_SUB = NPAD // NS
HALF = 40
E_SUB = 50176
NST = E_SUB // (STAGE * CHUNK)
N_ITERS = NST // 2
E_PAD = E_SUB * NS


def _sc_agg_body(xpad_hbm, idx_hbm, out_hbm, idx_a, idx_b, rows0, rows1,
                 agg_sh, sem_r0, sem_r1, sem_s0, sem_s1, sem_ia, sem_ib):
    c = lax.axis_index("c")
    s = lax.axis_index("s")
    base = s * ROWS_PER_SUB
    pltpu.sync_copy(xpad_hbm.at[c, pl.ds(base, ROWS_PER_SUB), :],
                    agg_sh.at[pl.ds(base, ROWS_PER_SUB), :])
    plsc.subcore_barrier()

    table = xpad_hbm.at[c]
    rows = (rows0, rows1)
    sem_r = (sem_r0, sem_r1)
    sem_s = (sem_s0, sem_s1)

    def wait_rows(b):
        pltpu.make_async_copy(table.at[idx_a.at[0, 0]], rows[b],
                              sem_r[b]).wait()

    def wait_scat(b):
        pltpu.make_async_copy(rows[b], agg_sh.at[idx_a.at[1, 0]],
                              sem_s[b]).wait()

    def wait_idx(iv, sem):
        pltpu.make_async_copy(idx_hbm.at[s, 0], iv, sem).wait()

    pltpu.sync_copy(idx_hbm.at[s, 0], idx_a)
    pltpu.async_copy(idx_hbm.at[s, 1], idx_b, sem_ib)
    pltpu.async_copy(table.at[idx_a.at[0, 0]], rows0, sem_r0)

    def body(i, carry):
        st_a = jnp.minimum(2 * i + 2, NST - 1)
        st_b = jnp.minimum(2 * i + 3, NST - 1)
        @pl.when(i > 0)
        def _():
            wait_scat(1)
        pltpu.async_copy(table.at[idx_a.at[0, 1]], rows1, sem_r1)
        wait_rows(0)
        pltpu.async_copy(rows0, agg_sh.at[idx_a.at[1, 0]], sem_s0, add=True)
        wait_idx(idx_b, sem_ib)
        wait_scat(0)
        pltpu.async_copy(table.at[idx_b.at[0, 0]], rows0, sem_r0)
        wait_rows(1)
        pltpu.async_copy(rows1, agg_sh.at[idx_a.at[1, 1]], sem_s1, add=True)
        pltpu.async_copy(idx_hbm.at[s, st_a], idx_a, sem_ia)
        wait_scat(1)
        pltpu.async_copy(table.at[idx_b.at[0, 1]], rows1, sem_r1)
        wait_rows(0)
        pltpu.async_copy(rows0, agg_sh.at[idx_b.at[1, 0]], sem_s0, add=True)
        wait_idx(idx_a, sem_ia)
        wait_scat(0)
        pltpu.async_copy(table.at[idx_a.at[0, 0]], rows0, sem_r0)
        wait_rows(1)
        pltpu.async_copy(rows1, agg_sh.at[idx_b.at[1, 1]], sem_s1, add=True)
        pltpu.async_copy(idx_hbm.at[s, st_b], idx_b, sem_ib)
        return carry

    lax.fori_loop(0, N_ITERS, body, 0)
    wait_rows(0)
    wait_scat(1)
    wait_idx(idx_b, sem_ib)
    plsc.subcore_barrier()
    pltpu.sync_copy(agg_sh.at[pl.ds(base, ROWS_PER_SUB), :],
                    out_hbm.at[c, pl.ds(base, ROWS_PER_SUB), :])


@functools.cache
def _sc_agg():
    return pl.kernel(
        _sc_agg_body,
        out_type=jax.ShapeDtypeStruct((NC, NPAD, HALF), jnp.float32),
        mesh=plsc.VectorSubcoreMesh(core_axis_name="c", subcore_axis_name="s",
                                    num_cores=NC, num_subcores=NS),
        scratch_types=[
            pltpu.VMEM((2, STAGE, CHUNK), jnp.int32),
            pltpu.VMEM((2, STAGE, CHUNK), jnp.int32),
            pltpu.VMEM((CHUNK, HALF), jnp.float32),
            pltpu.VMEM((CHUNK, HALF), jnp.float32),
            pltpu.VMEM_SHARED((NPAD, HALF), jnp.float32),
            pltpu.SemaphoreType.DMA,
            pltpu.SemaphoreType.DMA,
            pltpu.SemaphoreType.DMA,
            pltpu.SemaphoreType.DMA,
            pltpu.SemaphoreType.DMA,
            pltpu.SemaphoreType.DMA,
        ],
        compiler_params=pltpu.CompilerParams(use_tc_tiling_on_sc=False),
    )


def _tc_mlp_pool_body(lo_ref, hi_ref, bcol_ref, w1a_ref, w1b_ref, b1_ref,
                      w2_ref, b2_ref, out_ref, cnt_ref):
    i = pl.program_id(0)
    nsteps = pl.num_programs(0)

    @pl.when(i == 0)
    def _init():
        out_ref[...] = jnp.zeros_like(out_ref)
        cnt_ref[...] = jnp.zeros_like(cnt_ref)

    lo = lo_ref[0].astype(jnp.bfloat16)
    hi = hi_ref[0].astype(jnp.bfloat16)
    h1 = jnp.dot(lo, w1a_ref[...], preferred_element_type=jnp.float32)
    h1 += jnp.dot(hi, w1b_ref[...], preferred_element_type=jnp.float32)
    h1 = jnp.maximum(h1 + b1_ref[...], 0.0).astype(jnp.bfloat16)
    h2 = jnp.dot(h1, w2_ref[...], preferred_element_type=jnp.float32)
    h2 += b2_ref[...]

    row = i * BLK + lax.broadcasted_iota(jnp.int32, (BLK, 1), 0)
    h2 = jnp.where(row < N_NODES, h2, 0.0).astype(jnp.bfloat16)

    b = bcol_ref[...]
    gids = lax.broadcasted_iota(jnp.int32, (BLK, N_GRAPHS), 1)
    onehot = jnp.where((b == gids) & (row < N_NODES), 1.0, 0.0).astype(
        jnp.bfloat16)
    out_ref[...] += lax.dot_general(onehot, h2, (((0,), (0,)), ((), ())),
                                    preferred_element_type=jnp.float32)
    cnt_ref[...] += lax.dot_general(onehot, jnp.ones((BLK, 8), jnp.bfloat16),
                                    (((0,), (0,)), ((), ())),
                                    preferred_element_type=jnp.float32)

    @pl.when(i == nsteps - 1)
    def _final():
        counts = jnp.maximum(cnt_ref[:, 0:1], 1.0)
        out_ref[...] = out_ref[...] / counts


def _tc_mlp_pool(agg3d, bcol, w1a, w1b, b1, w2, b2):
    grid = (NPAD + BLK - 1) // BLK
    return pl.pallas_call(
        _tc_mlp_pool_body,
        grid=(grid,),
        in_specs=[
            pl.BlockSpec((1, BLK, HALF), lambda i: (0, i, 0)),
            pl.BlockSpec((1, BLK, HALF), lambda i: (1, i, 0)),
            pl.BlockSpec((BLK, 1), lambda i: (i, 0)),
            pl.BlockSpec((HALF, HIDDEN), lambda i: (0, 0)),
            pl.BlockSpec((HALF, HIDDEN), lambda i: (0, 0)),
            pl.BlockSpec((1, HIDDEN), lambda i: (0, 0)),
            pl.BlockSpec((HIDDEN, HIDDEN), lambda i: (0, 0)),
            pl.BlockSpec((1, HIDDEN), lambda i: (0, 0)),
        ],
        out_specs=pl.BlockSpec((N_GRAPHS, HIDDEN), lambda i: (0, 0)),
        out_shape=jax.ShapeDtypeStruct((N_GRAPHS, HIDDEN), jnp.float32),
        scratch_shapes=[pltpu.VMEM((N_GRAPHS, 8), jnp.float32)],
    )(agg3d, agg3d, bcol, w1a, w1b, b1, w2, b2)


def kernel(x, edge_index, batch, W1, b1, W2, b2):
    src = edge_index[0].astype(jnp.int32)
    dst = edge_index[1].astype(jnp.int32)

    xa = jnp.zeros((NPAD, HALF), jnp.float32).at[:N_NODES, :].set(x[:, :HALF])
    xb = jnp.zeros((NPAD, HALF), jnp.float32).at[:N_NODES, :IN_DIM - HALF].set(
        x[:, HALF:])
    xpad = jnp.stack([xa, xb])

    pad = jnp.full((E_PAD - N_EDGES,), N_NODES, jnp.int32)
    src_p = jnp.concatenate([src, pad]).reshape(NS, NST, 1, STAGE, CHUNK)
    dst_p = jnp.concatenate([dst, pad]).reshape(NS, NST, 1, STAGE, CHUNK)
    idx_all = jnp.concatenate([src_p, dst_p], axis=2)

    agg3d = _sc_agg()(xpad, idx_all)

    bcol = jnp.full((NPAD, 1), N_GRAPHS, jnp.int32).at[:N_NODES, 0].set(
        batch.astype(jnp.int32))

    w1a = W1[:HALF, :].astype(jnp.bfloat16)
    w1b = jnp.zeros((HALF, HIDDEN), jnp.float32).at[:IN_DIM - HALF, :].set(
        W1[HALF:, :]).astype(jnp.bfloat16)
    return _tc_mlp_pool(agg3d, bcol, w1a, w1b, b1.reshape(1, HIDDEN),
                        W2.astype(jnp.bfloat16), b2.reshape(1, HIDDEN))
